# baseline (speedup 1.0000x reference)
import jax
import jax.numpy as jnp
from jax.experimental import pallas as pl

_BN = 2048


def _add_body(x_ref, pos_ref, o_ref):
    o_ref[...] = x_ref[...] + pos_ref[...]


@jax.jit
def kernel(x, token_embedding):
    B, N, D = x.shape
    nb = N // _BN
    out = pl.pallas_call(
        _add_body,
        grid=(nb, B),
        in_specs=[
            pl.BlockSpec((_BN, D), lambda n, b: (b * nb + n, 0)),
            pl.BlockSpec((_BN, D), lambda n, b: (n, 0)),
        ],
        out_specs=pl.BlockSpec((_BN, D), lambda n, b: (b * nb + n, 0)),
        out_shape=jax.ShapeDtypeStruct((B * N, D), x.dtype),
    )(x.reshape(B * N, D), token_embedding)
    return out.reshape(B, N, D)
